# baseline (speedup 1.0000x reference)
import functools

import numpy as np
import jax
import jax.numpy as jnp
from jax import lax
from jax.experimental import pallas as pl
from jax.experimental.pallas import tpu as pltpu
from jax.experimental.pallas import tpu_sc as plsc


def _sc_info():
    try:
        info = plsc.get_sparse_core_info()
        return info.num_cores, info.num_subcores
    except Exception:
        return 2, 16


@functools.lru_cache(maxsize=None)
def _build(B, S, V, D):
    NC, NS = _sc_info()
    NW = NC * NS
    assert S % NW == 0
    P = S // NW
    C = 32
    assert P % C == 0
    PH = P // C
    NCHUNK = PH * B
    assert D % 16 == 0
    KV = D // 16
    scale = float(np.sqrt(np.float32(D)))

    mesh = plsc.VectorSubcoreMesh(core_axis_name="c", subcore_axis_name="s")

    @functools.partial(
        pl.kernel,
        out_type=jax.ShapeDtypeStruct((B * S, D), jnp.float32),
        mesh=mesh,
        scratch_types=[
            pltpu.VMEM((B, P), jnp.int32),
            pltpu.VMEM((2, C, D), jnp.float32),
            pltpu.VMEM((C, D), jnp.float32),
            pltpu.SemaphoreType.DMA,
            pltpu.SemaphoreType.DMA,
            pltpu.SemaphoreType.DMA,
            pltpu.SemaphoreType.DMA,
        ],
    )
    def emb_kernel(seq_hbm, table_hbm, pe_hbm, out_hbm,
                   idx_v, buf, pebuf, g0, g1, s0, s1):
        wid = lax.axis_index("s") * NC + lax.axis_index("c")
        wpos = wid * P
        gsem = (g0, g1)
        ssem = (s0, s1)

        for b in range(B):
            pltpu.sync_copy(seq_hbm.at[b, pl.ds(wpos, P)], idx_v.at[b])
        pltpu.sync_copy(pe_hbm.at[pl.ds(wpos, C)], pebuf)

        def issue_gather(j):
            ph, b = divmod(j, B)
            p = j % 2
            return pltpu.async_copy(
                table_hbm.at[idx_v.at[b, pl.ds(ph * C, C)]],
                buf.at[p], gsem[p])

        def compute_and_store(j, gat):
            ph, b = divmod(j, B)
            p = j % 2
            gat.wait()

            @plsc.parallel_loop(0, C)
            def _rows(r):
                @plsc.parallel_loop(0, KV, unroll=8)
                def _cols(k):
                    buf[p, r, pl.ds(k * 16, 16)] = (
                        buf[p, r, pl.ds(k * 16, 16)] * scale
                        + pebuf[r, pl.ds(k * 16, 16)])

            return pltpu.async_copy(
                buf.at[p], out_hbm.at[pl.ds(b * S + wpos + ph * C, C)],
                ssem[p])

        gats = {0: issue_gather(0)}
        stores = {}
        for j in range(1, NCHUNK + 1):
            if j < NCHUNK:
                if j - 2 in stores:
                    stores.pop(j - 2).wait()
                gats[j] = issue_gather(j)
            jj = j - 1
            if jj > 0 and jj % B == 0:
                pltpu.sync_copy(pe_hbm.at[pl.ds(wpos + (jj // B) * C, C)],
                                pebuf)
            stores[jj] = compute_and_store(jj, gats.pop(jj))
        for st in stores.values():
            st.wait()

    return emb_kernel


def _pe_runtime(S, D, sequences):
    zero = (sequences[0, 0] * 0).astype(jnp.float32)
    k = (jnp.arange(D) // 2).astype(jnp.float32) * 2.0
    inv_freq = jnp.exp(-k * (float(np.log(10000.0)) / D))
    pos = jnp.arange(S, dtype=jnp.float32)[:, None] + zero
    theta = pos * inv_freq[None, :]
    return jnp.where((jnp.arange(D) % 2) == 0, jnp.sin(theta), jnp.cos(theta))


def kernel(sequences, table):
    B, S = sequences.shape
    V, D = table.shape
    pe = _pe_runtime(S, D, sequences)
    emb_kernel = _build(B, S, V, D)
    out = emb_kernel(sequences.astype(jnp.int32), table, pe)
    return out.reshape(B, S, D)


# speedup vs baseline: 1.4127x; 1.4127x over previous
import functools

import numpy as np
import jax
import jax.numpy as jnp
from jax import lax
from jax.experimental import pallas as pl
from jax.experimental.pallas import tpu as pltpu
from jax.experimental.pallas import tpu_sc as plsc


def _sc_info():
    try:
        info = plsc.get_sparse_core_info()
        return info.num_cores, info.num_subcores
    except Exception:
        return 2, 16


@functools.lru_cache(maxsize=None)
def _build(B, S, V, D):
    NC, NS = _sc_info()
    NW = NC * NS
    assert S % NW == 0
    P = S // NW
    C = 32
    assert P % C == 0
    PH = P // C
    NCHUNK = PH * B
    assert D % 16 == 0
    KV = D // 16
    scale = float(np.sqrt(np.float32(D)))

    mesh = plsc.VectorSubcoreMesh(core_axis_name="c", subcore_axis_name="s")

    @functools.partial(
        pl.kernel,
        out_type=jax.ShapeDtypeStruct((B * S, D), jnp.float32),
        mesh=mesh,
        scratch_types=[
            pltpu.VMEM((B, P), jnp.int32),
            pltpu.VMEM((2, C, D), jnp.float32),
            pltpu.VMEM((C, D), jnp.float32),
            pltpu.SemaphoreType.DMA,
            pltpu.SemaphoreType.DMA,
            pltpu.SemaphoreType.DMA,
            pltpu.SemaphoreType.DMA,
        ],
    )
    def emb_kernel(seq_hbm, table_hbm, pe_hbm, out_hbm,
                   idx_v, buf, pebuf, g0, g1, s0, s1):
        wid = lax.axis_index("s") * NC + lax.axis_index("c")
        wpos = wid * P
        gsem = (g0, g1)
        ssem = (s0, s1)

        for b in range(B):
            pltpu.sync_copy(seq_hbm.at[b, pl.ds(wpos, P)], idx_v.at[b])
        pltpu.sync_copy(pe_hbm.at[pl.ds(wpos, C)], pebuf)

        def issue_gather(j):
            ph, b = divmod(j, B)
            p = j % 2
            return pltpu.async_copy(
                table_hbm.at[idx_v.at[b, pl.ds(ph * C, C)]],
                buf.at[p], gsem[p])

        def compute_and_store(j, gat):
            ph, b = divmod(j, B)
            p = j % 2
            gat.wait()

            @plsc.parallel_loop(0, C)
            def _rows(r):
                @plsc.parallel_loop(0, KV, unroll=8)
                def _cols(k):
                    buf[p, r, pl.ds(k * 16, 16)] = (
                        buf[p, r, pl.ds(k * 16, 16)] * scale
                        + pebuf[r, pl.ds(k * 16, 16)])

            return pltpu.async_copy(
                buf.at[p], out_hbm.at[pl.ds(b * S + wpos + ph * C, C)],
                ssem[p])

        gats = {0: issue_gather(0)}
        stores = {}
        for j in range(1, NCHUNK + 1):
            if j < NCHUNK:
                if j - 2 in stores:
                    stores.pop(j - 2).wait()
                gats[j] = issue_gather(j)
            jj = j - 1
            if jj > 0 and jj % B == 0:
                pltpu.sync_copy(pe_hbm.at[pl.ds(wpos + (jj // B) * C, C)],
                                pebuf)
            stores[jj] = compute_and_store(jj, gats.pop(jj))
        for st in stores.values():
            st.wait()

    return emb_kernel


@functools.lru_cache(maxsize=None)
def _pe_tables_np(S, D, NQ):
    NR = S // NQ
    d = np.arange(D, dtype=np.float64)
    w = np.power(10000.0, -(d - d % 2) / np.float32(D))
    even = (np.arange(D) % 2) == 0
    a = (np.arange(NQ, dtype=np.float64)[:, None] * NR) * w[None, :]
    b = np.arange(NR, dtype=np.float64)[:, None] * w[None, :]
    p1 = np.where(even[None, :], np.sin(a), np.cos(a))
    p2 = np.where(even[None, :], np.cos(a), -np.sin(a))
    q1 = np.cos(b)
    q2 = np.sin(b)
    return (p1.astype(np.float32), p2.astype(np.float32),
            q1.astype(np.float32), q2.astype(np.float32))


def _pe_runtime(S, D, sequences):
    NQ = 32
    p1, p2, q1, q2 = (jnp.asarray(t) for t in _pe_tables_np(S, D, NQ))
    zero = (sequences[0, 0] * 0).astype(jnp.float32)
    pe3 = ((p1[:, None, :] + zero) * q1[None, :, :]
           + p2[:, None, :] * q2[None, :, :])
    return pe3.reshape(S, D)


def kernel(sequences, table):
    B, S = sequences.shape
    V, D = table.shape
    pe = _pe_runtime(S, D, sequences)
    emb_kernel = _build(B, S, V, D)
    out = emb_kernel(sequences.astype(jnp.int32), table, pe)
    return out.reshape(B, S, D)


# speedup vs baseline: 1.6251x; 1.1503x over previous
import functools

import numpy as np
import jax
import jax.numpy as jnp
from jax import lax
from jax.experimental import pallas as pl
from jax.experimental.pallas import tpu as pltpu
from jax.experimental.pallas import tpu_sc as plsc


def _sc_info():
    try:
        info = plsc.get_sparse_core_info()
        return info.num_cores, info.num_subcores
    except Exception:
        return 2, 16


@functools.lru_cache(maxsize=None)
def _build(B, S, V, D):
    NC, NS = _sc_info()
    NW = NC * NS
    assert S % NW == 0
    P = S // NW
    CP = 32
    C = 16
    NB = 4
    AHEAD = NB - 1
    assert P % CP == 0 and CP % C == 0
    NWIN = P // CP
    HP = CP // C
    NCHUNK = NWIN * B * HP
    assert D % 16 == 0
    KV = D // 16
    scale = float(np.sqrt(np.float32(D)))

    def coords(j):
        w0, t = divmod(j, B * HP)
        b, h = divmod(t, HP)
        return w0, b, h

    mesh = plsc.VectorSubcoreMesh(core_axis_name="c", subcore_axis_name="s")

    @functools.partial(
        pl.kernel,
        out_type=jax.ShapeDtypeStruct((B * S, D), jnp.float32),
        mesh=mesh,
        scratch_types=[
            pltpu.VMEM((B, P), jnp.int32),
            pltpu.VMEM((NB, C, D), jnp.float32),
            pltpu.VMEM((CP, D), jnp.float32),
            pltpu.SemaphoreType.DMA((NB,)),
            pltpu.SemaphoreType.DMA((NB,)),
            pltpu.SemaphoreType.DMA,
        ],
    )
    def emb_kernel(seq_hbm, table_hbm, pe_hbm, out_hbm,
                   idx_v, buf, pebuf, gsem, ssem, psem):
        wid = lax.axis_index("s") * NC + lax.axis_index("c")
        wpos = wid * P

        for b in range(B):
            pltpu.sync_copy(seq_hbm.at[b, pl.ds(wpos, P)], idx_v.at[b])

        def issue_pe(w0):
            return pltpu.async_copy(
                pe_hbm.at[pl.ds(wpos + w0 * CP, CP)], pebuf, psem)

        def issue_gather(j):
            w0, b, h = coords(j)
            p = j % NB
            return pltpu.async_copy(
                table_hbm.at[idx_v.at[b, pl.ds(w0 * CP + h * C, C)]],
                buf.at[p], gsem.at[p])

        def compute(j):
            w0, b, h = coords(j)
            p = j % NB

            @plsc.parallel_loop(0, C)
            def _rows(r):
                @plsc.parallel_loop(0, KV, unroll=8)
                def _cols(k):
                    buf[p, r, pl.ds(k * 16, 16)] = (
                        buf[p, r, pl.ds(k * 16, 16)] * scale
                        + pebuf[h * C + r, pl.ds(k * 16, 16)])

        def issue_store(j):
            w0, b, h = coords(j)
            p = j % NB
            return pltpu.async_copy(
                buf.at[p],
                out_hbm.at[pl.ds(b * S + wpos + w0 * CP + h * C, C)],
                ssem.at[p])

        pe_wait = issue_pe(0)
        gats = {j: issue_gather(j) for j in range(AHEAD)}
        stores = {}
        for j in range(NCHUNK):
            w0, _, _ = coords(j)
            if pe_wait is not None and (j == 0 or coords(j - 1)[0] != w0):
                pe_wait.wait()
                pe_wait = None
            gats.pop(j).wait()
            compute(j)
            if j + 1 < NCHUNK and coords(j + 1)[0] != w0:
                pe_wait = issue_pe(w0 + 1)
            stores[j] = issue_store(j)
            nj = j + AHEAD
            if nj < NCHUNK:
                if nj - NB in stores:
                    stores.pop(nj - NB).wait()
                gats[nj] = issue_gather(nj)
        for st in stores.values():
            st.wait()

    return emb_kernel


@functools.lru_cache(maxsize=None)
def _pe_tables_np(S, D, NQ):
    NR = S // NQ
    d = np.arange(D, dtype=np.float64)
    w = np.power(10000.0, -(d - d % 2) / np.float32(D))
    even = (np.arange(D) % 2) == 0
    a = (np.arange(NQ, dtype=np.float64)[:, None] * NR) * w[None, :]
    b = np.arange(NR, dtype=np.float64)[:, None] * w[None, :]
    p1 = np.where(even[None, :], np.sin(a), np.cos(a))
    p2 = np.where(even[None, :], np.cos(a), -np.sin(a))
    q1 = np.cos(b)
    q2 = np.sin(b)
    return (p1.astype(np.float32), p2.astype(np.float32),
            q1.astype(np.float32), q2.astype(np.float32))


def _pe_runtime(S, D, sequences):
    NQ = 32
    p1, p2, q1, q2 = (jnp.asarray(t) for t in _pe_tables_np(S, D, NQ))
    zero = (sequences[0, 0] * 0).astype(jnp.float32)
    pe3 = ((p1[:, None, :] + zero) * q1[None, :, :]
           + p2[:, None, :] * q2[None, :, :])
    return pe3.reshape(S, D)


def kernel(sequences, table):
    B, S = sequences.shape
    V, D = table.shape
    pe = _pe_runtime(S, D, sequences)
    emb_kernel = _build(B, S, V, D)
    out = emb_kernel(sequences.astype(jnp.int32), table, pe)
    return out.reshape(B, S, D)
